# Initial kernel scaffold; baseline (speedup 1.0000x reference)
import jax
import jax.numpy as jnp
from jax.experimental import pallas as pl


def kernel(input_ids, input_embeds, new_embeds):
    raise NotImplementedError("write your pallas kernel here")


# speedup vs baseline: 4.1508x; 4.1508x over previous
import functools

import jax
import jax.numpy as jnp
from jax import lax
from jax.experimental import pallas as pl
from jax.experimental.pallas import tpu as pltpu
from jax.experimental.pallas import tpu_sc as plsc

VOCAB = 100000
EMBED_DIM = 64
SOFT_PROMPT_LEN = 128
BATCH = 4096
HIST = 200

NC = 2
NS = 16
NW = NC * NS
B_TOTAL = BATCH * HIST
BPW = B_TOTAL // NW
RPB = 128
NB = BPW // RPB
DEPTH = 8


def _emb_kernel(tbl, ids, out, idx_v, rows_v, gsem, ssem):
    wid = lax.axis_index("s") * NC + lax.axis_index("c")

    pltpu.sync_copy(ids.at[wid], idx_v)

    def fire_gather(g, b):
        pltpu.async_copy(tbl.at[idx_v.at[g]], rows_v.at[b], gsem.at[b])

    def wait_gather(g, b):
        pltpu.make_async_copy(tbl.at[idx_v.at[g]], rows_v.at[b], gsem.at[b]).wait()

    def fire_store(g, b):
        pltpu.async_copy(rows_v.at[b], out.at[wid, g], ssem.at[b])

    def wait_store(g, b):
        pltpu.make_async_copy(rows_v.at[b], out.at[wid, g], ssem.at[b]).wait()

    for b in range(DEPTH - 1):
        fire_gather(b, b)

    def outer(t, carry):
        for b in range(DEPTH):
            g = t * DEPTH + b
            wait_gather(g, b)
            fire_store(g, b)
            bm1 = (b - 1) % DEPTH

            @pl.when((g >= 1) & (g + DEPTH - 1 < NB))
            def _():
                wait_store(g - 1, bm1)

            @pl.when(g + DEPTH - 1 < NB)
            def _():
                fire_gather(g + DEPTH - 1, bm1)

        return carry

    lax.fori_loop(0, NB // DEPTH, outer, 0)

    for b in range(DEPTH):
        wait_store(NB - DEPTH + b, (NB - DEPTH + b) % DEPTH)


@functools.partial(
    pl.kernel,
    out_type=jax.ShapeDtypeStruct((NW, NB, RPB, EMBED_DIM), jnp.float32),
    mesh=plsc.VectorSubcoreMesh(
        core_axis_name="c", subcore_axis_name="s", num_cores=NC, num_subcores=NS
    ),
    scratch_types=[
        pltpu.VMEM((NB, RPB), jnp.int32),
        pltpu.VMEM((DEPTH, RPB, EMBED_DIM), jnp.float32),
        pltpu.SemaphoreType.DMA((DEPTH,)),
        pltpu.SemaphoreType.DMA((DEPTH,)),
    ],
    compiler_params=pltpu.CompilerParams(use_tc_tiling_on_sc=False),
)
def _emb_call(tbl, ids, out, idx_v, rows_v, gsem, ssem):
    _emb_kernel(tbl, ids, out, idx_v, rows_v, gsem, ssem)


def kernel(input_ids, input_embeds, new_embeds):
    tbl = jnp.concatenate([input_embeds, new_embeds], axis=0)
    ids = input_ids.reshape(NW, NB, RPB).astype(jnp.int32)
    out = _emb_call(tbl, ids)
    return out.reshape(BATCH, HIST, EMBED_DIM)
